# baseline (speedup 1.0000x reference)
import jax
import jax.numpy as jnp
from jax.experimental import pallas as pl
from jax.experimental.pallas import tpu as pltpu

_B, _S, _D_IN, _D_OUT, _E, _R = 4, 2048, 2048, 2048, 8, 8
_SCALING = 16.0 / 8.0
_BS = 512
_BO = 512


def _fused_body(ids_ref, x_ref, w_ref, bias_ref, a_ref, bb_ref, o_ref):
    x = x_ref[0]
    w = w_ref[...]
    a = a_ref[0]
    bb = bb_ref[0]
    dn = (((1,), (0,)), ((), ()))
    base = jax.lax.dot_general(x, w, dn, preferred_element_type=jnp.float32)
    lr = jax.lax.dot_general(x, a, dn, preferred_element_type=jnp.float32)
    delta = jax.lax.dot_general(lr, bb, dn, preferred_element_type=jnp.float32)
    o_ref[0] = base + delta * _SCALING + bias_ref[...]


def kernel(hidden_states, adapter_ids, W, b, lora_a, lora_b):
    ids = adapter_ids.astype(jnp.int32)
    bias2 = b.reshape(1, _D_OUT)
    grid_spec = pltpu.PrefetchScalarGridSpec(
        num_scalar_prefetch=1,
        grid=(_B, _D_OUT // _BO, _S // _BS),
        in_specs=[
            pl.BlockSpec((1, _BS, _D_IN), lambda bi, oi, si, ids: (bi, si, 0)),
            pl.BlockSpec((_D_IN, _BO), lambda bi, oi, si, ids: (0, oi)),
            pl.BlockSpec((1, _BO), lambda bi, oi, si, ids: (0, oi)),
            pl.BlockSpec((1, _D_IN, _R), lambda bi, oi, si, ids: (ids[bi], 0, 0)),
            pl.BlockSpec((1, _R, _BO), lambda bi, oi, si, ids: (ids[bi], 0, oi)),
        ],
        out_specs=pl.BlockSpec((1, _BS, _BO), lambda bi, oi, si, ids: (bi, si, oi)),
    )
    return pl.pallas_call(
        _fused_body,
        grid_spec=grid_spec,
        out_shape=jax.ShapeDtypeStruct((_B, _S, _D_OUT), jnp.float32),
    )(ids, hidden_states, W, bias2, lora_a, lora_b)


# speedup vs baseline: 1.5083x; 1.5083x over previous
import jax
import jax.numpy as jnp
from jax.experimental import pallas as pl
from jax.experimental.pallas import tpu as pltpu

_B, _S, _D_IN, _D_OUT, _E, _R = 4, 2048, 2048, 2048, 8, 8
_SCALING = 16.0 / 8.0
_BS = 512


def _fused_body(ids_ref, x_ref, w_ref, bias_ref, a_ref, bb_ref, o_ref):
    x = x_ref[0]
    w = w_ref[...]
    a = a_ref[0]
    bb = bb_ref[0]
    dn = (((1,), (0,)), ((), ()))
    base = jax.lax.dot_general(x, w, dn, preferred_element_type=jnp.float32)
    lr = jax.lax.dot_general(x, a, dn, preferred_element_type=jnp.float32)
    delta = jax.lax.dot_general(lr, bb, dn, preferred_element_type=jnp.float32)
    o_ref[0] = base + delta + bias_ref[...]


def kernel(hidden_states, adapter_ids, W, b, lora_a, lora_b):
    ids = adapter_ids.astype(jnp.int32)
    bias2 = b.reshape(1, _D_OUT)
    bb_scaled = lora_b * _SCALING
    grid_spec = pltpu.PrefetchScalarGridSpec(
        num_scalar_prefetch=1,
        grid=(_B, _S // _BS),
        in_specs=[
            pl.BlockSpec((1, _BS, _D_IN), lambda bi, si, ids: (bi, si, 0)),
            pl.BlockSpec((_D_IN, _D_OUT), lambda bi, si, ids: (0, 0)),
            pl.BlockSpec((1, _D_OUT), lambda bi, si, ids: (0, 0)),
            pl.BlockSpec((1, _D_IN, _R), lambda bi, si, ids: (ids[bi], 0, 0)),
            pl.BlockSpec((1, _R, _D_OUT), lambda bi, si, ids: (ids[bi], 0, 0)),
        ],
        out_specs=pl.BlockSpec((1, _BS, _D_OUT), lambda bi, si, ids: (bi, si, 0)),
    )
    return pl.pallas_call(
        _fused_body,
        grid_spec=grid_spec,
        out_shape=jax.ShapeDtypeStruct((_B, _S, _D_OUT), jnp.float32),
    )(ids, hidden_states, W, bias2, lora_a, bb_scaled)
